# baseline (speedup 1.0000x reference)
import functools

import jax
import jax.numpy as jnp
from jax import lax
from jax.experimental import pallas as pl
from jax.experimental.pallas import tpu as pltpu
from jax.experimental.pallas import tpu_sc as plsc

NC = 2
NS = 16
NW = NC * NS
L = 16
W = 128
CKW = 64
CHS = 256


def kernel(mem, idx, val, pred):
    M, D = mem.shape
    B = idx.shape[0]
    R = W // D
    Mw = M // R
    Bw = B // R
    per_w = (Mw // (NW * 2 * CKW)) * (2 * CKW)
    per = per_w * R
    span = M - (NW - 1) * per
    npairs = per_w // (2 * CKW)
    tail_w = Mw - NW * per_w
    n_groups = B // L

    mesh = plsc.VectorSubcoreMesh(core_axis_name="c", subcore_axis_name="s")

    @functools.partial(
        pl.kernel,
        out_type=jax.ShapeDtypeStruct((Mw, W), mem.dtype),
        mesh=mesh,
        compiler_params=pltpu.CompilerParams(needs_layout_passes=False,
                                             skip_device_barrier=True),
        scratch_types=[
            pltpu.VMEM((B,), jnp.int32),
            pltpu.VMEM((span,), jnp.int32),
            pltpu.VMEM((B + L,), jnp.int32),
            pltpu.VMEM((B + L,), jnp.int32),
            pltpu.VMEM((L,), jnp.int32),
            pltpu.VMEM((CHS // 4, W), jnp.float32),
            pltpu.VMEM((CKW, W), jnp.float32),
            pltpu.VMEM((CKW, W), jnp.float32),
            pltpu.SemaphoreType.DMA,
            pltpu.SemaphoreType.DMA,
            pltpu.SemaphoreType.DMA,
            pltpu.SemaphoreType.DMA,
            pltpu.SemaphoreType.DMA,
            pltpu.SemaphoreType.DMA,
        ],
    )
    def _buf(mem_h, idx_h, val_h, pred_h, out_h,
             idx_v, tag_v, mb_v, mr_v, pred_s, stage_v, cb0, cb1,
             isem0, isem1, osem0, osem1, gsem, ssem):
        wid = lax.axis_index("s") * NC + lax.axis_index("c")
        lo_w = wid * per_w
        lo = lo_w * R
        is_last = wid == NW - 1

        cbufs = (cb0, cb1)
        isems = (isem0, isem1)
        osems = (osem0, osem1)

        def in_cp(g, b):
            return pltpu.make_async_copy(
                mem_h.at[pl.ds(lo_w + g * CKW, CKW), :], cbufs[b], isems[b])

        def out_cp(g, b):
            return pltpu.make_async_copy(
                cbufs[b], out_h.at[pl.ds(lo_w + g * CKW, CKW), :], osems[b])

        in_cp(jnp.int32(0), 0).start()

        def copy_body(p, carry):
            g0 = p * 2
            in_cp(g0, 0).wait()

            @pl.when(p > 0)
            def _w1():
                out_cp(g0, 1).wait()
            in_cp(g0 + 1, 1).start()
            out_cp(g0, 0).start()
            in_cp(g0 + 1, 1).wait()

            @pl.when(p + 1 < npairs)
            def _w0():
                out_cp(g0, 0).wait()
                in_cp(g0 + 2, 0).start()
            out_cp(g0 + 1, 1).start()
            return carry

        lax.fori_loop(0, npairs, copy_body, jnp.int32(0))
        out_cp(jnp.int32(0), 0).wait()
        out_cp(jnp.int32(0), 1).wait()

        if tail_w:
            @pl.when(is_last)
            def _tail():
                toff = per_w * NW
                left = tail_w
                off = 0
                while left:
                    csz = min(CKW, left)
                    pltpu.sync_copy(mem_h.at[pl.ds(toff + off, csz), :],
                                    cb0.at[pl.ds(0, csz), :])
                    pltpu.sync_copy(cb0.at[pl.ds(0, csz), :],
                                    out_h.at[pl.ds(toff + off, csz), :])
                    off += csz
                    left -= csz

        pltpu.sync_copy(pred_h, pred_s.at[pl.ds(0, 1)])
        pltpu.sync_copy(idx_h, idx_v)
        p_ok = pred_s[pl.ds(0, L)][0] != 0
        hi0 = lax.select(is_last, jnp.int32(M), jnp.int32(lo + per))
        hi = lax.select(p_ok, hi0, jnp.int32(lo))
        ngrp = lax.select(is_last, jnp.int32(span // L), jnp.int32(per // L))
        _ones = jnp.full((L,), 1, jnp.int32)
        _zeros = jnp.zeros((L,), jnp.int32)
        _neg = jnp.full((L,), -1, jnp.int32)

        def clear_body(j, carry):
            tag_v[pl.ds(j * L, L)] = _neg
            return carry

        lax.fori_loop(0, ngrp, clear_body, jnp.int32(0))

        def scan_body(i, carry):
            v = idx_v[pl.ds(i * L, L)]
            b_vec = lax.iota(jnp.int32, L) + i * L
            m = (v >= lo) & (v < hi)
            plsc.store_scatter(tag_v, [v - lo], b_vec, mask=m)
            return carry

        lax.fori_loop(0, n_groups, scan_body, jnp.int32(0))

        def gather_body(j, cur):
            t = tag_v[pl.ds(j * L, L)]
            m2 = t >= 0
            rows_vec = lax.iota(jnp.int32, L) + (lo + j * L)
            c = plsc.cumsum(jnp.where(m2, _ones, _zeros))
            posn = cur + c - 1
            plsc.store_scatter(mb_v, [posn], t, mask=m2)
            plsc.store_scatter(mr_v, [posn], rows_vec, mask=m2)
            return cur + c[L - 1]

        n = lax.fori_loop(0, ngrp, gather_body, jnp.int32(0))

        def chunk_body(c, carry):
            base = c * CHS
            cnt = lax.min(jnp.int32(CHS), n - base)

            def gfire(k, c2):
                b = mb_v[pl.ds(base + k, L)][0]
                pltpu.async_copy(
                    val_h.at[pl.ds(b // 4, 1), pl.ds((b % 4) * D, D)],
                    stage_v.at[pl.ds(k // 4, 1), pl.ds((k % 4) * D, D)],
                    gsem)
                return c2

            lax.fori_loop(0, cnt, gfire, jnp.int32(0))

            def gdrain(k, c2):
                pltpu.make_async_copy(
                    val_h.at[pl.ds(0, 1), pl.ds(0, D)],
                    stage_v.at[pl.ds(0, 1), pl.ds(0, D)], gsem).wait()
                return c2

            lax.fori_loop(0, cnt, gdrain, jnp.int32(0))

            def sfire(k, c2):
                r = mr_v[pl.ds(base + k, L)][0]
                pltpu.async_copy(
                    stage_v.at[pl.ds(k // 4, 1), pl.ds((k % 4) * D, D)],
                    out_h.at[pl.ds(r // 4, 1), pl.ds((r % 4) * D, D)],
                    ssem)
                return c2

            lax.fori_loop(0, cnt, sfire, jnp.int32(0))

            def sdrain(k, c2):
                pltpu.make_async_copy(
                    stage_v.at[pl.ds(0, 1), pl.ds(0, D)],
                    out_h.at[pl.ds(0, 1), pl.ds(0, D)], ssem).wait()
                return c2

            lax.fori_loop(0, cnt, sdrain, jnp.int32(0))
            return carry

        nchk = (n + (CHS - 1)) // CHS
        lax.fori_loop(0, nchk, chunk_body, jnp.int32(0))

    out_w = _buf(mem.reshape(Mw, W), idx, val.reshape(Bw, W),
                 pred.astype(jnp.int32))
    return out_w.reshape(M, D)


# speedup vs baseline: 1.1206x; 1.1206x over previous
import functools

import jax
import jax.numpy as jnp
from jax import lax
from jax.experimental import pallas as pl
from jax.experimental.pallas import tpu as pltpu
from jax.experimental.pallas import tpu_sc as plsc

NC = 2
NS = 16
NW = NC * NS
L = 16
CKR = 128
CHS = 64
UNR = 8


def kernel(mem, idx, val, pred):
    M, D = mem.shape
    B = idx.shape[0]
    per = (M // (NW * 2 * CKR)) * (2 * CKR)
    span = M - (NW - 1) * per
    npairs = per // (2 * CKR)
    tail = span - per
    n_groups = B // L

    mesh = plsc.VectorSubcoreMesh(core_axis_name="c", subcore_axis_name="s")

    @functools.partial(
        pl.kernel,
        out_type=jax.ShapeDtypeStruct((M, D), mem.dtype),
        mesh=mesh,
        compiler_params=pltpu.CompilerParams(needs_layout_passes=False,
                                             skip_device_barrier=True),
        scratch_types=[
            pltpu.VMEM((B,), jnp.int32),
            pltpu.VMEM((span,), jnp.int32),
            pltpu.VMEM((B + L,), jnp.int32),
            pltpu.VMEM((B + L,), jnp.int32),
            pltpu.VMEM((L,), jnp.int32),
            pltpu.VMEM((CHS, 32), jnp.float32),
            pltpu.VMEM((CKR, 32), jnp.float32),
            pltpu.VMEM((CKR, 32), jnp.float32),
            pltpu.SemaphoreType.DMA,
            pltpu.SemaphoreType.DMA,
            pltpu.SemaphoreType.DMA,
            pltpu.SemaphoreType.DMA,
            pltpu.SemaphoreType.DMA,
            pltpu.SemaphoreType.DMA,
        ],
    )
    def _buf(mem_h, idx_h, val_h, pred_h, out_h,
             idx_v, tag_v, mb_v, mr_v, pred_s, stage_v, cb0, cb1,
             isem0, isem1, osem0, osem1, gsem, ssem):
        wid = lax.axis_index("s") * NC + lax.axis_index("c")
        lo = wid * per
        is_last = wid == NW - 1

        cbufs = (cb0, cb1)
        isems = (isem0, isem1)
        osems = (osem0, osem1)

        def in_cp(g, b):
            return pltpu.make_async_copy(
                mem_h.at[pl.ds(lo + g * CKR, CKR), :], cbufs[b], isems[b])

        def out_cp(g, b):
            return pltpu.make_async_copy(
                cbufs[b], out_h.at[pl.ds(lo + g * CKR, CKR), :], osems[b])

        in_cp(jnp.int32(0), 0).start()

        def copy_body(p, carry):
            g0 = p * 2
            in_cp(g0, 0).wait()

            @pl.when(p > 0)
            def _w1():
                out_cp(g0, 1).wait()
            in_cp(g0 + 1, 1).start()
            out_cp(g0, 0).start()
            in_cp(g0 + 1, 1).wait()

            @pl.when(p + 1 < npairs)
            def _w0():
                out_cp(g0, 0).wait()
                in_cp(g0 + 2, 0).start()
            out_cp(g0 + 1, 1).start()
            return carry

        lax.fori_loop(0, npairs, copy_body, jnp.int32(0))
        out_cp(jnp.int32(0), 0).wait()
        out_cp(jnp.int32(0), 1).wait()

        if tail:
            @pl.when(is_last)
            def _tail():
                toff = per * NW
                left = tail
                off = 0
                while left:
                    csz = min(CKR, left)
                    pltpu.sync_copy(mem_h.at[pl.ds(toff + off, csz), :],
                                    cb0.at[pl.ds(0, csz), :])
                    pltpu.sync_copy(cb0.at[pl.ds(0, csz), :],
                                    out_h.at[pl.ds(toff + off, csz), :])
                    off += csz
                    left -= csz

        pltpu.sync_copy(pred_h, pred_s.at[pl.ds(0, 1)])
        pltpu.sync_copy(idx_h, idx_v)
        p_ok = pred_s[pl.ds(0, L)][0] != 0
        hi0 = lax.select(is_last, jnp.int32(M), jnp.int32(lo + per))
        hi = lax.select(p_ok, hi0, jnp.int32(lo))
        _ones = jnp.full((L,), 1, jnp.int32)
        _zeros = jnp.zeros((L,), jnp.int32)
        _neg = jnp.full((L,), -1, jnp.int32)

        def clear_body(j, carry):
            tag_v[pl.ds(j * L, L)] = _neg
            return carry

        lax.fori_loop(0, per // L, clear_body, jnp.int32(0), unroll=UNR)
        if span != per:
            @pl.when(is_last)
            def _clear_tail():
                lax.fori_loop(per // L, span // L, clear_body, jnp.int32(0),
                              unroll=UNR)

        def scan_body(i, carry):
            v = idx_v[pl.ds(i * L, L)]
            b_vec = lax.iota(jnp.int32, L) + i * L
            m = (v >= lo) & (v < hi)
            plsc.store_scatter(tag_v, [v - lo], b_vec, mask=m)
            return carry

        lax.fori_loop(0, n_groups, scan_body, jnp.int32(0), unroll=UNR)

        def gather_body(j, cur):
            t = tag_v[pl.ds(j * L, L)]
            m2 = t >= 0
            rows_vec = lax.iota(jnp.int32, L) + (lo + j * L)
            c = plsc.cumsum(jnp.where(m2, _ones, _zeros))
            posn = cur + c - 1
            plsc.store_scatter(mb_v, [posn], t, mask=m2)
            plsc.store_scatter(mr_v, [posn], rows_vec, mask=m2)
            return cur + c[L - 1]

        n0 = lax.fori_loop(0, per // L, gather_body, jnp.int32(0),
                           unroll=UNR)
        if span != per:
            n = lax.cond(is_last,
                         lambda a: lax.fori_loop(per // L, span // L,
                                                 gather_body, a, unroll=UNR),
                         lambda a: a, n0)
        else:
            n = n0

        def chunk_body(c, carry):
            base = c * CHS
            cnt = lax.min(jnp.int32(CHS), n - base)

            def gfire(k, c2):
                b = mb_v[pl.ds(base + k, L)][0]
                pltpu.async_copy(val_h.at[pl.ds(b, 1), :],
                                 stage_v.at[pl.ds(k, 1), :], gsem)
                return c2

            lax.fori_loop(0, cnt, gfire, jnp.int32(0))

            def gdrain(k, c2):
                pltpu.make_async_copy(val_h.at[pl.ds(0, 1), :],
                                      stage_v.at[pl.ds(0, 1), :], gsem).wait()
                return c2

            lax.fori_loop(0, cnt, gdrain, jnp.int32(0))

            def sfire(k, c2):
                r = mr_v[pl.ds(base + k, L)][0]
                pltpu.async_copy(stage_v.at[pl.ds(k, 1), :],
                                 out_h.at[pl.ds(r, 1), :], ssem)
                return c2

            lax.fori_loop(0, cnt, sfire, jnp.int32(0))

            def sdrain(k, c2):
                pltpu.make_async_copy(stage_v.at[pl.ds(0, 1), :],
                                      out_h.at[pl.ds(0, 1), :], ssem).wait()
                return c2

            lax.fori_loop(0, cnt, sdrain, jnp.int32(0))
            return carry

        nchk = (n + (CHS - 1)) // CHS
        lax.fori_loop(0, nchk, chunk_body, jnp.int32(0))

    return _buf(mem, idx, val, pred.astype(jnp.int32))


# speedup vs baseline: 1.1286x; 1.0071x over previous
import functools

import jax
import jax.numpy as jnp
from jax import lax
from jax.experimental import pallas as pl
from jax.experimental.pallas import tpu as pltpu
from jax.experimental.pallas import tpu_sc as plsc

NC = 2
NS = 16
NW = NC * NS
L = 16
CKR = 128
CHS = 64
UNR = 8


def kernel(mem, idx, val, pred):
    M, D = mem.shape
    B = idx.shape[0]
    per = (M // (NW * 2 * CKR)) * (2 * CKR)
    span = M - (NW - 1) * per
    npairs = per // (2 * CKR)
    tail = span - per
    n_groups = B // L

    mesh = plsc.VectorSubcoreMesh(core_axis_name="c", subcore_axis_name="s")

    @functools.partial(
        pl.kernel,
        out_type=jax.ShapeDtypeStruct((M, D), mem.dtype),
        mesh=mesh,
        compiler_params=pltpu.CompilerParams(needs_layout_passes=False,
                                             skip_device_barrier=True),
        scratch_types=[
            pltpu.VMEM((B,), jnp.int32),
            pltpu.VMEM((span,), jnp.int32),
            pltpu.VMEM((B + L,), jnp.int32),
            pltpu.VMEM((B + L,), jnp.int32),
            pltpu.VMEM((L,), jnp.int32),
            pltpu.VMEM((CHS, 32), jnp.float32),
            pltpu.VMEM((CKR, 32), jnp.float32),
            pltpu.VMEM((CKR, 32), jnp.float32),
            pltpu.SemaphoreType.DMA,
            pltpu.SemaphoreType.DMA,
            pltpu.SemaphoreType.DMA,
            pltpu.SemaphoreType.DMA,
            pltpu.SemaphoreType.DMA,
            pltpu.SemaphoreType.DMA,
        ],
    )
    def _buf(mem_h, idx_h, val_h, pred_h, out_h,
             idx_v, tag_v, mb_v, mr_v, pred_s, stage_v, cb0, cb1,
             isem0, isem1, osem0, osem1, gsem, ssem):
        wid = lax.axis_index("s") * NC + lax.axis_index("c")
        lo = wid * per
        is_last = wid == NW - 1

        cbufs = (cb0, cb1)
        isems = (isem0, isem1)
        osems = (osem0, osem1)

        def in_cp(g, b):
            return pltpu.make_async_copy(
                mem_h.at[pl.ds(lo + g * CKR, CKR), :], cbufs[b], isems[b])

        def out_cp(g, b):
            return pltpu.make_async_copy(
                cbufs[b], out_h.at[pl.ds(lo + g * CKR, CKR), :], osems[b])

        in_cp(jnp.int32(0), 0).start()
        pltpu.sync_copy(pred_h, pred_s.at[pl.ds(0, 1)])
        pltpu.sync_copy(idx_h, idx_v)
        p_ok = pred_s[pl.ds(0, L)][0] != 0
        hi0 = lax.select(is_last, jnp.int32(M), jnp.int32(lo + per))
        hi = lax.select(p_ok, hi0, jnp.int32(lo))
        _ones = jnp.full((L,), 1, jnp.int32)
        _zeros = jnp.zeros((L,), jnp.int32)
        _neg = jnp.full((L,), -1, jnp.int32)
        half = npairs // 2
        ncl = (per // L + half - 1) // half
        nsc = (n_groups + (npairs - half) - 1) // (npairs - half)

        def clear_body(j, carry):
            tag_v[pl.ds(j * L, L)] = _neg
            return carry

        def scan_body(i, carry):
            v = idx_v[pl.ds(i * L, L)]
            b_vec = lax.iota(jnp.int32, L) + i * L
            m = (v >= lo) & (v < hi)
            plsc.store_scatter(tag_v, [v - lo], b_vec, mask=m)
            return carry

        def copy_body(p, carry):
            g0 = p * 2
            in_cp(g0, 0).wait()

            @pl.when(p > 0)
            def _w1():
                out_cp(g0, 1).wait()
            in_cp(g0 + 1, 1).start()
            out_cp(g0, 0).start()
            in_cp(g0 + 1, 1).wait()

            @pl.when(p + 1 < npairs)
            def _w0():
                out_cp(g0, 0).wait()
                in_cp(g0 + 2, 0).start()
            out_cp(g0 + 1, 1).start()

            @pl.when(p < half)
            def _cl():
                def b1(k, c3):
                    j = p * ncl + k
                    @pl.when(j < per // L)
                    def _():
                        clear_body(j, c3)
                    return c3
                lax.fori_loop(0, ncl, b1, jnp.int32(0), unroll=ncl)
                if span != per:
                    @pl.when((p == half - 1) & is_last)
                    def _ctail():
                        lax.fori_loop(per // L, span // L, clear_body,
                                      jnp.int32(0), unroll=UNR)

            @pl.when(p >= half)
            def _sc():
                def b2(k, c3):
                    i = (p - half) * nsc + k
                    @pl.when(i < n_groups)
                    def _():
                        scan_body(i, c3)
                    return c3
                lax.fori_loop(0, nsc, b2, jnp.int32(0), unroll=nsc)
            return carry

        lax.fori_loop(0, npairs, copy_body, jnp.int32(0))
        out_cp(jnp.int32(0), 0).wait()
        out_cp(jnp.int32(0), 1).wait()

        if tail:
            @pl.when(is_last)
            def _tail():
                toff = per * NW
                left = tail
                off = 0
                while left:
                    csz = min(CKR, left)
                    pltpu.sync_copy(mem_h.at[pl.ds(toff + off, csz), :],
                                    cb0.at[pl.ds(0, csz), :])
                    pltpu.sync_copy(cb0.at[pl.ds(0, csz), :],
                                    out_h.at[pl.ds(toff + off, csz), :])
                    off += csz
                    left -= csz


        def gather_body(j, cur):
            t = tag_v[pl.ds(j * L, L)]
            m2 = t >= 0
            rows_vec = lax.iota(jnp.int32, L) + (lo + j * L)
            c = plsc.cumsum(jnp.where(m2, _ones, _zeros))
            posn = cur + c - 1
            plsc.store_scatter(mb_v, [posn], t, mask=m2)
            plsc.store_scatter(mr_v, [posn], rows_vec, mask=m2)
            return cur + c[L - 1]

        n0 = lax.fori_loop(0, per // L, gather_body, jnp.int32(0),
                           unroll=UNR)
        if span != per:
            n = lax.cond(is_last,
                         lambda a: lax.fori_loop(per // L, span // L,
                                                 gather_body, a, unroll=UNR),
                         lambda a: a, n0)
        else:
            n = n0

        def chunk_body(c, carry):
            base = c * CHS
            cnt = lax.min(jnp.int32(CHS), n - base)

            def gfire(k, c2):
                b = mb_v[pl.ds(base + k, L)][0]
                pltpu.async_copy(val_h.at[pl.ds(b, 1), :],
                                 stage_v.at[pl.ds(k, 1), :], gsem)
                return c2

            lax.fori_loop(0, cnt, gfire, jnp.int32(0))

            def gdrain(k, c2):
                pltpu.make_async_copy(val_h.at[pl.ds(0, 1), :],
                                      stage_v.at[pl.ds(0, 1), :], gsem).wait()
                return c2

            lax.fori_loop(0, cnt, gdrain, jnp.int32(0))

            def sfire(k, c2):
                r = mr_v[pl.ds(base + k, L)][0]
                pltpu.async_copy(stage_v.at[pl.ds(k, 1), :],
                                 out_h.at[pl.ds(r, 1), :], ssem)
                return c2

            lax.fori_loop(0, cnt, sfire, jnp.int32(0))

            def sdrain(k, c2):
                pltpu.make_async_copy(stage_v.at[pl.ds(0, 1), :],
                                      out_h.at[pl.ds(0, 1), :], ssem).wait()
                return c2

            lax.fori_loop(0, cnt, sdrain, jnp.int32(0))
            return carry

        nchk = (n + (CHS - 1)) // CHS
        lax.fori_loop(0, nchk, chunk_body, jnp.int32(0))

    return _buf(mem, idx, val, pred.astype(jnp.int32))
